# baseline (speedup 1.0000x reference)
import functools

import jax
import jax.numpy as jnp
from jax.experimental import pallas as pl
from jax.experimental.pallas import tpu as pltpu

N_NODES = 8192
N_DOCS = 16384
HIDDEN = 256
DOC = 64
K = 16

BM = 256
BR = 128
NEG = -3e38


def _ln(x, w, b):
    m = jnp.mean(x, axis=-1, keepdims=True)
    v = jnp.var(x, axis=-1, keepdims=True)
    return (x - m) / jnp.sqrt(v + 1e-5) * w + b


def _gelu(x):
    return 0.5 * x * (1.0 + jax.lax.erf(x * 0.7071067811865476))


PREC = jax.lax.Precision.DEFAULT


def _dot_t(a, b):
    return jax.lax.dot_general(a, b, (((1,), (1,)), ((), ())),
                               preferred_element_type=jnp.float32,
                               precision=PREC)


def _dot(a, b):
    return jax.lax.dot_general(a, b, (((1,), (0,)), ((), ())),
                               preferred_element_type=jnp.float32,
                               precision=PREC)


def _layer_kernel(a_ref, h_ref, w_ref, lnw_ref, lnb_ref, o_ref):
    z = _dot(a_ref[...], h_ref[...])
    z = _dot_t(z, w_ref[...])
    o_ref[...] = _gelu(_ln(z, lnw_ref[...], lnb_ref[...]))


def _gcn_layer(A, H, W, lnw, lnb):
    n = A.shape[0]
    grid = (n // BM,)
    return pl.pallas_call(
        _layer_kernel,
        grid=grid,
        in_specs=[
            pl.BlockSpec((BM, n), lambda i: (i, 0)),
            pl.BlockSpec((n, HIDDEN), lambda i: (0, 0)),
            pl.BlockSpec((HIDDEN, HIDDEN), lambda i: (0, 0)),
            pl.BlockSpec((1, HIDDEN), lambda i: (0, 0)),
            pl.BlockSpec((1, HIDDEN), lambda i: (0, 0)),
        ],
        out_specs=pl.BlockSpec((BM, HIDDEN), lambda i: (i, 0)),
        out_shape=jax.ShapeDtypeStruct((n, HIDDEN), jnp.float32),
    )(A, H, W, lnw, lnb)


NCHUNK = 1024


def _retrieval_kernel(h_ref, doct_ref, doc_ref, pw1_ref, pb1_ref, pw2_ref,
                      pb2_ref, gwh_ref, gwc_ref, gb_ref, cw_ref, cb_ref,
                      o_ref, t_ref):
    h = h_ref[...]
    q = _gelu(_dot_t(h, pw1_ref[...]) + pb1_ref[...])
    q = _dot_t(q, pw2_ref[...]) + pb2_ref[...]
    s = _dot(q, doct_ref[...]) * 0.125

    s3 = s.reshape(BR, N_DOCS // NCHUNK, NCHUNK)
    cm1 = jnp.max(s3, axis=1)
    cm2 = jnp.max(jnp.where(s3 < cm1[:, None, :], s3, NEG), axis=1)
    cm3 = jnp.max(jnp.where(s3 < cm2[:, None, :], s3, NEG), axis=1)
    cur, nxt, nx2 = cm1, cm2, cm3
    m = jnp.max(cur, axis=1, keepdims=True)
    v0 = m
    for _ in range(K - 1):
        hit = cur == m
        cur = jnp.where(hit, nxt, cur)
        nxt = jnp.where(hit, nx2, nxt)
        nx2 = jnp.where(hit, NEG, nx2)
        m = jnp.max(cur, axis=1, keepdims=True)
    cnt = jnp.sum((s >= m).astype(jnp.float32), axis=1, keepdims=True)
    t_ref[...] = jnp.broadcast_to(m, t_ref.shape)

    @pl.when(jnp.logical_not(jnp.all(cnt == float(K))))
    def _fallback():
        mm = v0
        for _ in range(K - 1):
            mm = jnp.max(jnp.where(s < mm, s, NEG), axis=1, keepdims=True)
        t_ref[...] = jnp.broadcast_to(mm, t_ref.shape)

    t = t_ref[:, :1]
    w = jnp.where(s >= t, jnp.exp((s - v0) * 2.0), 0.0)
    denom = jnp.sum(w, axis=1, keepdims=True)
    ctx = _dot(w, doc_ref[...]) / denom

    g = jax.nn.sigmoid(_dot_t(h, gwh_ref[...]) + _dot_t(ctx, gwc_ref[...])
                       + gb_ref[...])
    ctx_t = _dot_t(ctx, cw_ref[...]) + cb_ref[...]
    o_ref[...] = g * h + (1.0 - g) * ctx_t


def _retrieval(H, docT, doc, pw1, pb1, pw2, pb2, gwh, gwc, gb, cw, cb):
    n = H.shape[0]
    grid = (n // BR,)
    c = lambda i: (0, 0)
    return pl.pallas_call(
        _retrieval_kernel,
        grid=grid,
        in_specs=[
            pl.BlockSpec((BR, HIDDEN), lambda i: (i, 0)),
            pl.BlockSpec((DOC, N_DOCS), c),
            pl.BlockSpec((N_DOCS, DOC), c),
            pl.BlockSpec((HIDDEN, HIDDEN), c),
            pl.BlockSpec((1, HIDDEN), c),
            pl.BlockSpec((DOC, HIDDEN), c),
            pl.BlockSpec((1, DOC), c),
            pl.BlockSpec((HIDDEN, HIDDEN), c),
            pl.BlockSpec((HIDDEN, DOC), c),
            pl.BlockSpec((1, HIDDEN), c),
            pl.BlockSpec((HIDDEN, DOC), c),
            pl.BlockSpec((1, HIDDEN), c),
        ],
        out_specs=pl.BlockSpec((BR, HIDDEN), lambda i: (i, 0)),
        out_shape=jax.ShapeDtypeStruct((n, HIDDEN), jnp.float32),
        scratch_shapes=[pltpu.VMEM((BR, 128), jnp.float32)],
    )(H, docT, doc, pw1, pb1, pw2, pb2, gwh, gwc, gb, cw, cb)


@jax.jit
def kernel(A, X, doc_emb, W1, W2, W3, ln1_w, ln1_b, ln2_w, ln2_b, ln3_w,
           ln3_b, proj_w1, proj_b1, proj_w2, proj_b2, gate_w, gate_b,
           ctx_w, ctx_b):
    r = lambda v: v.reshape(1, -1)
    H = _gcn_layer(A, X, W1, r(ln1_w), r(ln1_b))
    H = _gcn_layer(A, H, W2, r(ln2_w), r(ln2_b))
    H = _gcn_layer(A, H, W3, r(ln3_w), r(ln3_b))
    docT = doc_emb.T
    gwh = gate_w[:, :HIDDEN]
    gwc = gate_w[:, HIDDEN:]
    return _retrieval(H, docT, doc_emb, proj_w1, r(proj_b1), proj_w2,
                      r(proj_b2), gwh, gwc, r(gate_b), ctx_w, r(ctx_b))


# speedup vs baseline: 3.8117x; 3.8117x over previous
import functools

import jax
import jax.numpy as jnp
from jax.experimental import pallas as pl
from jax.experimental.pallas import tpu as pltpu

N_NODES = 8192
N_DOCS = 16384
HIDDEN = 256
DOC = 64
K = 16

BM = 256
BR = 128
NEG = -3e38


def _ln(x, w, b):
    m = jnp.mean(x, axis=-1, keepdims=True)
    v = jnp.var(x, axis=-1, keepdims=True)
    return (x - m) / jnp.sqrt(v + 1e-5) * w + b


def _gelu(x):
    return 0.5 * x * (1.0 + jax.lax.erf(x * 0.7071067811865476))


PREC = jax.lax.Precision.DEFAULT


def _dot_t(a, b):
    return jax.lax.dot_general(a, b, (((1,), (1,)), ((), ())),
                               preferred_element_type=jnp.float32,
                               precision=PREC)


def _dot(a, b):
    return jax.lax.dot_general(a, b, (((1,), (0,)), ((), ())),
                               preferred_element_type=jnp.float32,
                               precision=PREC)


def _layer_kernel(a_ref, h_ref, w_ref, lnw_ref, lnb_ref, o_ref):
    z = _dot(a_ref[...], h_ref[...])
    z = _dot_t(z, w_ref[...])
    o_ref[...] = _gelu(_ln(z, lnw_ref[...], lnb_ref[...]))


def _gcn_layer(A, H, W, lnw, lnb):
    n = A.shape[0]
    grid = (n // BM,)
    return pl.pallas_call(
        _layer_kernel,
        grid=grid,
        in_specs=[
            pl.BlockSpec((BM, n), lambda i: (i, 0)),
            pl.BlockSpec((n, HIDDEN), lambda i: (0, 0)),
            pl.BlockSpec((HIDDEN, HIDDEN), lambda i: (0, 0)),
            pl.BlockSpec((1, HIDDEN), lambda i: (0, 0)),
            pl.BlockSpec((1, HIDDEN), lambda i: (0, 0)),
        ],
        out_specs=pl.BlockSpec((BM, HIDDEN), lambda i: (i, 0)),
        out_shape=jax.ShapeDtypeStruct((n, HIDDEN), jnp.float32),
    )(A, H, W, lnw, lnb)


NCHUNK = 1024


def _retrieval_kernel(h_ref, doct_ref, doc_ref, pw1_ref, pb1_ref, pw2_ref,
                      pb2_ref, gwh_ref, gwc_ref, gb_ref, cw_ref, cb_ref,
                      o_ref, t_ref):
    h = h_ref[...]
    q = _gelu(_dot_t(h, pw1_ref[...]) + pb1_ref[...])
    q = _dot_t(q, pw2_ref[...]) + pb2_ref[...]
    s = _dot(q, doct_ref[...]) * 0.125

    cm1 = s[:, :NCHUNK]
    cm2 = jnp.full_like(cm1, NEG)
    cm3 = cm2
    for j in range(1, N_DOCS // NCHUNK):
        x = s[:, j * NCHUNK:(j + 1) * NCHUNK]
        a = jnp.minimum(cm1, x)
        cm1 = jnp.maximum(cm1, x)
        b = jnp.minimum(cm2, a)
        cm2 = jnp.maximum(cm2, a)
        cm3 = jnp.maximum(cm3, b)
    cur, nxt, nx2 = cm1, cm2, cm3
    m = jnp.max(cur, axis=1, keepdims=True)
    v0 = m
    for _ in range(K - 1):
        hit = cur == m
        cur = jnp.where(hit, nxt, cur)
        nxt = jnp.where(hit, nx2, nxt)
        nx2 = jnp.where(hit, NEG, nx2)
        m = jnp.max(cur, axis=1, keepdims=True)
    cnt = jnp.sum((s >= m).astype(jnp.float32), axis=1, keepdims=True)
    t_ref[...] = jnp.broadcast_to(m, t_ref.shape)

    @pl.when(jnp.logical_not(jnp.all(cnt == float(K))))
    def _fallback():
        mm = v0
        for _ in range(K - 1):
            mm = jnp.max(jnp.where(s < mm, s, NEG), axis=1, keepdims=True)
        t_ref[...] = jnp.broadcast_to(mm, t_ref.shape)

    t = t_ref[:, :1]
    w = jnp.where(s >= t, jnp.exp((s - v0) * 2.0), 0.0)
    denom = jnp.sum(w, axis=1, keepdims=True)
    ctx = _dot(w, doc_ref[...]) / denom

    g = jax.nn.sigmoid(_dot_t(h, gwh_ref[...]) + _dot_t(ctx, gwc_ref[...])
                       + gb_ref[...])
    ctx_t = _dot_t(ctx, cw_ref[...]) + cb_ref[...]
    o_ref[...] = g * h + (1.0 - g) * ctx_t


def _retrieval(H, docT, doc, pw1, pb1, pw2, pb2, gwh, gwc, gb, cw, cb):
    n = H.shape[0]
    grid = (n // BR,)
    c = lambda i: (0, 0)
    return pl.pallas_call(
        _retrieval_kernel,
        grid=grid,
        in_specs=[
            pl.BlockSpec((BR, HIDDEN), lambda i: (i, 0)),
            pl.BlockSpec((DOC, N_DOCS), c),
            pl.BlockSpec((N_DOCS, DOC), c),
            pl.BlockSpec((HIDDEN, HIDDEN), c),
            pl.BlockSpec((1, HIDDEN), c),
            pl.BlockSpec((DOC, HIDDEN), c),
            pl.BlockSpec((1, DOC), c),
            pl.BlockSpec((HIDDEN, HIDDEN), c),
            pl.BlockSpec((HIDDEN, DOC), c),
            pl.BlockSpec((1, HIDDEN), c),
            pl.BlockSpec((HIDDEN, DOC), c),
            pl.BlockSpec((1, HIDDEN), c),
        ],
        out_specs=pl.BlockSpec((BR, HIDDEN), lambda i: (i, 0)),
        out_shape=jax.ShapeDtypeStruct((n, HIDDEN), jnp.float32),
        scratch_shapes=[pltpu.VMEM((BR, 128), jnp.float32)],
    )(H, docT, doc, pw1, pb1, pw2, pb2, gwh, gwc, gb, cw, cb)


@jax.jit
def kernel(A, X, doc_emb, W1, W2, W3, ln1_w, ln1_b, ln2_w, ln2_b, ln3_w,
           ln3_b, proj_w1, proj_b1, proj_w2, proj_b2, gate_w, gate_b,
           ctx_w, ctx_b):
    r = lambda v: v.reshape(1, -1)
    H = _gcn_layer(A, X, W1, r(ln1_w), r(ln1_b))
    H = _gcn_layer(A, H, W2, r(ln2_w), r(ln2_b))
    H = _gcn_layer(A, H, W3, r(ln3_w), r(ln3_b))
    docT = doc_emb.T
    gwh = gate_w[:, :HIDDEN]
    gwc = gate_w[:, HIDDEN:]
    return _retrieval(H, docT, doc_emb, proj_w1, r(proj_b1), proj_w2,
                      r(proj_b2), gwh, gwc, r(gate_b), ctx_w, r(ctx_b))
